# baseline (speedup 1.0000x reference)
import functools

import jax
import jax.numpy as jnp
from jax import lax
from jax.experimental import pallas as pl
from jax.experimental.pallas import tpu as pltpu
from jax.experimental.pallas import tpu_sc as plsc

_VOCAB = 1000000
_EMB = 64
_CTX = 16
_RF = 8
_B = 16384
_T = _CTX + _RF - 1


def _make_sc_gather(V, D, B):
    info = plsc.get_sparse_core_info()
    NC, NS = info.num_cores, info.num_subcores
    NW = NC * NS
    b_per_w = B // NW
    mesh = plsc.VectorSubcoreMesh(core_axis_name="c", subcore_axis_name="s")

    @functools.partial(
        pl.kernel,
        mesh=mesh,
        out_type=jax.ShapeDtypeStruct((B, D), jnp.float32),
        scratch_types=[
            pltpu.VMEM((b_per_w,), jnp.int32),
            pltpu.VMEM((b_per_w, D), jnp.float32),
            pltpu.SemaphoreType.DMA,
        ],
        compiler_params=pltpu.CompilerParams(use_tc_tiling_on_sc=False),
    )
    def gather_k(table_hbm, idx_hbm, out_hbm, idx_v, rows_v, sem):
        wid = lax.axis_index("s") * NC + lax.axis_index("c")
        base = wid * b_per_w
        pltpu.sync_copy(idx_hbm.at[pl.ds(base, b_per_w)], idx_v)
        pltpu.async_copy(table_hbm.at[idx_v], rows_v, sem).wait()
        pltpu.sync_copy(rows_v, out_hbm.at[pl.ds(base, b_per_w)])

    return gather_k


def _tc_body(state_ref, emb_ref, wc_ref, out_ref):
    s = state_ref[...]
    st = jnp.swapaxes(s, 1, 2)
    e = emb_ref[...]
    inv_rf = 1.0 / _RF
    ys = []
    for j in range(_CTX):
        x = e if j == 0 else st[:, j - 1, :]
        m = jnp.sum(st[:, j : j + _RF, :], axis=1) * inv_rf
        c = jnp.concatenate([x, m], axis=1)
        y = jnp.tanh(jnp.dot(c, wc_ref[j], preferred_element_type=jnp.float32))
        ys.append(y)
    Y = jnp.stack(ys, axis=1)
    outt = jnp.concatenate([Y, st[:, _CTX:, :]], axis=1)
    out_ref[...] = jnp.swapaxes(outt, 1, 2)


def _tc_call(network_state, emb, Wc, bs):
    n_blocks = _B // bs
    return pl.pallas_call(
        _tc_body,
        grid=(n_blocks,),
        in_specs=[
            pl.BlockSpec((bs, _EMB, _T), lambda i: (i, 0, 0)),
            pl.BlockSpec((bs, _EMB), lambda i: (i, 0)),
            pl.BlockSpec((_CTX, 2 * _EMB, _EMB), lambda i: (0, 0, 0)),
        ],
        out_specs=pl.BlockSpec((bs, _EMB, _T), lambda i: (i, 0, 0)),
        out_shape=jax.ShapeDtypeStruct((_B, _EMB, _T), jnp.float32),
        compiler_params=pltpu.CompilerParams(
            dimension_semantics=("arbitrary",),
        ),
    )(network_state, emb, Wc)


def kernel(emb_table, network_state, W1, W2, input_indices):
    idx = input_indices[:, 0]
    emb = _make_sc_gather(_VOCAB, _EMB, _B)(emb_table, idx)
    Wc = jnp.concatenate([W1, W2], axis=1)
    return _tc_call(network_state, emb, Wc, bs=128)
